# baseline (speedup 1.0000x reference)
import functools
import math

import numpy as np
import jax
import jax.numpy as jnp
from jax.experimental import pallas as pl
from jax.experimental.pallas import tpu as pltpu

_NEG = -3.4e38


@functools.lru_cache(maxsize=None)
def _cnt_transposed(L_Q: int, L_K: int, sample_k: int) -> np.ndarray:
    with jax.ensure_compile_time_eval():
        idx = np.asarray(
            jax.random.randint(jax.random.key(42), (L_Q, sample_k), 0, L_K)
        )
    cnt = np.zeros((L_K, L_Q), np.int8)
    np.add.at(cnt, (idx, np.arange(L_Q)[:, None]), 1)
    return cnt


def _make_body(L: int, D: int, U: int, KB: int, scale: float, HPB: int):
    def body(cnt_ref, q_ref, k_ref, v_ref, ctx_ref, attn_ref, oh_ref):
        lane = jax.lax.broadcasted_iota(jnp.int32, (1, L), 1)
        rowi = jax.lax.broadcasted_iota(jnp.int32, (U, L), 0)
        coli = jax.lax.broadcasted_iota(jnp.int32, (U, L), 1)

        Ms = []
        for hh in range(HPB):
            q = q_ref[hh]
            mx = jnp.full((1, L), _NEG, jnp.float32)
            sm = jnp.zeros((1, L), jnp.float32)
            for kb in range(L // KB):
                k_blk = k_ref[hh, kb * KB:(kb + 1) * KB, :]
                st = jax.lax.dot_general(
                    k_blk, q, (((1,), (1,)), ((), ())),
                    preferred_element_type=jnp.float32)
                cf = cnt_ref[kb * KB:(kb + 1) * KB, :].astype(jnp.float32)
                masked = jnp.where(cf > 0, st, _NEG)
                mx = jnp.maximum(mx, jnp.max(masked, axis=0, keepdims=True))
                sm = sm + jnp.sum(st * cf, axis=0, keepdims=True)
            Ms.append(mx - sm * (1.0 / L))

        for u in range(U):
            for hh in range(HPB):
                i0 = jnp.argmax(Ms[hh], axis=1).reshape(1, 1)
                sel = lane == i0
                oh_ref[hh, u:u + 1, :] = sel.astype(jnp.float32)
                Ms[hh] = jnp.where(sel, _NEG, Ms[hh])

        for hh in range(HPB):
            qr = jax.lax.dot_general(
                oh_ref[hh], q_ref[hh], (((1,), (0,)), ((), ())),
                preferred_element_type=jnp.float32)
            qk = jax.lax.dot_general(
                qr, k_ref[hh], (((1,), (1,)), ((), ())),
                preferred_element_type=jnp.float32)
            s = jnp.where(coli <= rowi, qk * scale, -1000000000.0)
            smax = jnp.max(s, axis=1, keepdims=True)
            e = jnp.exp(s - smax)
            attn = e / jnp.sum(e, axis=1, keepdims=True)
            attn_ref[hh] = attn
            ctx_ref[hh] = jax.lax.dot_general(
                attn, v_ref[hh], (((1,), (0,)), ((), ())),
                preferred_element_type=jnp.float32)

    return body


def kernel(queries, keys, values):
    B, L, H, D = queries.shape
    BH = B * H
    U = 5 * int(math.ceil(math.log(float(L))))
    scale = 1.0 / math.sqrt(D)
    KB = 512
    HPB = 4

    q = queries.reshape(BH, L, D)
    k = keys.reshape(BH, L, D)
    v = values.reshape(BH, L, D)
    cnt_t = jnp.asarray(_cnt_transposed(L, L, U))

    ctx, attn = pl.pallas_call(
        _make_body(L, D, U, KB, scale, HPB),
        grid=(BH // HPB,),
        in_specs=[
            pl.BlockSpec((L, L), lambda i: (0, 0)),
            pl.BlockSpec((HPB, L, D), lambda i: (i, 0, 0)),
            pl.BlockSpec((HPB, L, D), lambda i: (i, 0, 0)),
            pl.BlockSpec((HPB, L, D), lambda i: (i, 0, 0)),
        ],
        out_specs=[
            pl.BlockSpec((HPB, U, D), lambda i: (i, 0, 0)),
            pl.BlockSpec((HPB, U, L), lambda i: (i, 0, 0)),
        ],
        out_shape=[
            jax.ShapeDtypeStruct((BH, U, D), jnp.float32),
            jax.ShapeDtypeStruct((BH, U, L), jnp.float32),
        ],
        scratch_shapes=[pltpu.VMEM((HPB, U, L), jnp.float32)],
    )(cnt_t, q, k, v)

    return ctx.reshape(B, H, U, D), attn.reshape(B, H, U, L)
